# baseline (speedup 1.0000x reference)
import functools

import jax
import jax.numpy as jnp
from jax import lax
from jax.experimental import pallas as pl
from jax.experimental.pallas import tpu as pltpu
from jax.experimental.pallas import tpu_sc as plsc

S_K = 64
W_K = 128
BATCH = 16384
NC = 2
NS = 16
LANES = 16
NW = NC * NS
ROWS_PER_W = BATCH // NW
CHUNK = 128
NCHUNK = ROWS_PER_W // CHUNK
NSLOT = 2

_mesh = plsc.VectorSubcoreMesh(
    core_axis_name="c", subcore_axis_name="s", num_cores=NC, num_subcores=NS
)


@functools.partial(
    pl.kernel,
    out_type=jax.ShapeDtypeStruct((BATCH, W_K), jnp.float32),
    mesh=_mesh,
    scratch_types=[
        pltpu.VMEM((1, CHUNK), jnp.int32),
        pltpu.VMEM((CHUNK,), jnp.int32),
        pltpu.VMEM((CHUNK, W_K), jnp.float32),
        pltpu.SemaphoreType.DMA,
    ],
)
def _sc_wgather(xT, w_tab, wout, xb, idx2, wrows, sem_w):
    wid = lax.axis_index("s") * NC + lax.axis_index("c")
    for c in range(NCHUNK):
        base = wid * ROWS_PER_W + c * CHUNK
        pltpu.sync_copy(xT.at[pl.ds(2, 1), pl.ds(base, CHUNK)], xb)

        def widx_body(g, carry):
            idx2[pl.ds(g * LANES, LANES)] = xb[0, pl.ds(g * LANES, LANES)]
            return carry

        lax.fori_loop(0, CHUNK // LANES, widx_body, 0)
        pltpu.async_copy(w_tab.at[idx2], wrows, sem_w).wait()
        pltpu.sync_copy(wrows, wout.at[pl.ds(base, CHUNK), :])


_slot_scratch = [
    pltpu.VMEM((2, CHUNK + LANES), jnp.int32),
    pltpu.VMEM((CHUNK, S_K), jnp.float32),
    pltpu.VMEM((CHUNK, S_K), jnp.float32),
    pltpu.VMEM((CHUNK, W_K), jnp.float32),
    pltpu.SemaphoreType.DMA,
    pltpu.SemaphoreType.DMA,
    pltpu.SemaphoreType.DMA,
]


@functools.partial(
    pl.kernel,
    out_type=jax.ShapeDtypeStruct((BATCH,), jnp.float32),
    mesh=_mesh,
    scratch_types=_slot_scratch * NSLOT + [
        pltpu.VMEM((CHUNK,), jnp.float32),
    ],
)
def _sc_dot(xT, s_tab, r_tab, w_rows_hbm, out, *scratch):
    nper = len(_slot_scratch)
    slots = [scratch[i * nper:(i + 1) * nper] for i in range(NSLOT)]
    outv = scratch[NSLOT * nper]

    wid = lax.axis_index("s") * NC + lax.axis_index("c")
    lane = lax.iota(jnp.int32, LANES)

    _dnums = lax.GatherDimensionNumbers(
        offset_dims=(), collapsed_slice_dims=(0,), start_index_map=(0,)
    )

    def _lane_shuffle(v, idx):
        return lax.gather(
            v, idx[:, None], _dnums, slice_sizes=(1,),
            mode=lax.GatherScatterMode.PROMISE_IN_BOUNDS,
        )

    def issue(c, slot):
        xbuf, srows, rrows, wrows, sem_s, sem_r, sem_w = slot
        base = wid * ROWS_PER_W + c * CHUNK
        for col in range(2):
            pltpu.sync_copy(xT.at[pl.ds(col, 1), pl.ds(base, CHUNK)],
                            xbuf.at[pl.ds(col, 1), pl.ds(0, CHUNK)])
        pltpu.async_copy(w_rows_hbm.at[pl.ds(base, CHUNK), :], wrows, sem_w)

        def dma_body(g, carry):
            v0 = xbuf[0, pl.ds(g * LANES, LANES)]
            v1 = xbuf[1, pl.ds(g * LANES, LANES)]
            for l in range(LANES):
                j = g * LANES + l
                pltpu.async_copy(s_tab.at[pl.ds(v0[l], 1), :],
                                 srows.at[pl.ds(j, 1), :], sem_s)
                pltpu.async_copy(r_tab.at[pl.ds(v1[l], 1), :],
                                 rrows.at[pl.ds(j, 1), :], sem_r)
            return carry

        lax.fori_loop(0, CHUNK // LANES, dma_body, 0)

    def drain(slot):
        _, srows, rrows, wrows, sem_s, sem_r, sem_w = slot
        pltpu.make_async_copy(s_tab.at[pl.ds(0, CHUNK), :], srows,
                              sem_s).wait()
        pltpu.make_async_copy(r_tab.at[pl.ds(0, CHUNK), :], rrows,
                              sem_r).wait()
        pltpu.make_async_copy(w_rows_hbm.at[pl.ds(0, CHUNK), :], wrows,
                              sem_w).wait()

    def compute(c, slot):
        _, srows, rrows, wrows, _, _, _ = slot
        base = wid * ROWS_PER_W + c * CHUNK

        def group_body(g, carry):
            def row_body(j, acc_out):
                i = g * LANES + j
                acc = srows[i, pl.ds(0, LANES)] * wrows[i, pl.ds(0, LANES)]
                for k in range(1, S_K // LANES):
                    acc = acc + (srows[i, pl.ds(k * LANES, LANES)]
                                 * wrows[i, pl.ds(k * LANES, LANES)])
                for k in range(S_K // LANES):
                    acc = acc + (rrows[i, pl.ds(k * LANES, LANES)]
                                 * wrows[i, pl.ds(S_K + k * LANES, LANES)])
                for d in (8, 4, 2, 1):
                    acc = acc + _lane_shuffle(acc, lane ^ d)
                return jnp.where(lane == j, acc, acc_out)

            accs = lax.fori_loop(
                0, LANES, row_body, jnp.zeros((LANES,), jnp.float32)
            )
            outv[pl.ds(g * LANES, LANES)] = 1.0 / (1.0 + jnp.exp(-accs))
            return carry

        lax.fori_loop(0, CHUNK // LANES, group_body, 0)
        pltpu.sync_copy(outv, out.at[pl.ds(base, CHUNK)])

    issue(0, slots[0])
    for c in range(NCHUNK):
        if c + 1 < NCHUNK:
            issue(c + 1, slots[(c + 1) % NSLOT])
        drain(slots[c % NSLOT])
        compute(c, slots[c % NSLOT])


def kernel(X, s_embeds, r_embeds, w_embeds):
    xT = X.astype(jnp.int32).T
    w_rows = _sc_wgather(xT, w_embeds)
    return _sc_dot(xT, s_embeds, r_embeds, w_rows)


# speedup vs baseline: 1.0178x; 1.0178x over previous
import functools

import jax
import jax.numpy as jnp
from jax import lax
from jax.experimental import pallas as pl
from jax.experimental.pallas import tpu as pltpu
from jax.experimental.pallas import tpu_sc as plsc

S_K = 64
W_K = 128
BATCH = 16384
NC = 2
NS = 16
LANES = 16
NW = NC * NS
ROWS_PER_W = BATCH // NW
CHUNK = 128
NCHUNK = ROWS_PER_W // CHUNK
NSLOT = 2

_mesh = plsc.VectorSubcoreMesh(
    core_axis_name="c", subcore_axis_name="s", num_cores=NC, num_subcores=NS
)

_slot_scratch = [
    pltpu.VMEM((3, CHUNK + LANES), jnp.int32),
    pltpu.VMEM((CHUNK,), jnp.int32),
    pltpu.VMEM((CHUNK, S_K), jnp.float32),
    pltpu.VMEM((CHUNK, S_K), jnp.float32),
    pltpu.VMEM((CHUNK, W_K), jnp.float32),
    pltpu.SemaphoreType.DMA,
    pltpu.SemaphoreType.DMA,
    pltpu.SemaphoreType.DMA,
]


@functools.partial(
    pl.kernel,
    out_type=jax.ShapeDtypeStruct((BATCH,), jnp.float32),
    mesh=_mesh,
    scratch_types=_slot_scratch * NSLOT + [
        pltpu.VMEM((CHUNK,), jnp.float32),
    ],
)
def _sc_forward(xT, s_tab, r_tab, w_tab, out, *scratch):
    nper = len(_slot_scratch)
    slots = [scratch[i * nper:(i + 1) * nper] for i in range(NSLOT)]
    outv = scratch[NSLOT * nper]

    wid = lax.axis_index("s") * NC + lax.axis_index("c")
    lane = lax.iota(jnp.int32, LANES)

    _dnums = lax.GatherDimensionNumbers(
        offset_dims=(), collapsed_slice_dims=(0,), start_index_map=(0,)
    )

    def _lane_shuffle(v, idx):
        return lax.gather(
            v, idx[:, None], _dnums, slice_sizes=(1,),
            mode=lax.GatherScatterMode.PROMISE_IN_BOUNDS,
        )

    def issue(c, slot):
        xbuf, idx2, srows, rrows, wrows, sem_s, sem_r, sem_w = slot
        base = wid * ROWS_PER_W + c * CHUNK
        for col in range(3):
            pltpu.sync_copy(xT.at[pl.ds(col, 1), pl.ds(base, CHUNK)],
                            xbuf.at[pl.ds(col, 1), pl.ds(0, CHUNK)])

        def widx_body(g, carry):
            idx2[pl.ds(g * LANES, LANES)] = xbuf[2, pl.ds(g * LANES, LANES)]
            return carry

        lax.fori_loop(0, CHUNK // LANES, widx_body, 0)
        pltpu.async_copy(w_tab.at[idx2], wrows, sem_w)

        def dma_body(g, carry):
            v0 = xbuf[0, pl.ds(g * LANES, LANES)]
            v1 = xbuf[1, pl.ds(g * LANES, LANES)]
            for l in range(LANES):
                j = g * LANES + l
                pltpu.async_copy(s_tab.at[pl.ds(v0[l], 1), :],
                                 srows.at[pl.ds(j, 1), :], sem_s)
                pltpu.async_copy(r_tab.at[pl.ds(v1[l], 1), :],
                                 rrows.at[pl.ds(j, 1), :], sem_r)
            return carry

        lax.fori_loop(0, CHUNK // LANES, dma_body, 0)

    def drain(slot):
        _, _, srows, rrows, wrows, sem_s, sem_r, sem_w = slot
        pltpu.make_async_copy(s_tab.at[pl.ds(0, CHUNK), :], srows,
                              sem_s).wait()
        pltpu.make_async_copy(r_tab.at[pl.ds(0, CHUNK), :], rrows,
                              sem_r).wait()
        pltpu.make_async_copy(w_tab.at[pl.ds(0, CHUNK), :], wrows,
                              sem_w).wait()

    def compute(c, slot):
        _, _, srows, rrows, wrows, _, _, _ = slot
        base = wid * ROWS_PER_W + c * CHUNK

        def group_body(g, carry):
            def row_body(j, acc_out):
                i = g * LANES + j
                acc = srows[i, pl.ds(0, LANES)] * wrows[i, pl.ds(0, LANES)]
                for k in range(1, S_K // LANES):
                    acc = acc + (srows[i, pl.ds(k * LANES, LANES)]
                                 * wrows[i, pl.ds(k * LANES, LANES)])
                for k in range(S_K // LANES):
                    acc = acc + (rrows[i, pl.ds(k * LANES, LANES)]
                                 * wrows[i, pl.ds(S_K + k * LANES, LANES)])
                for d in (8, 4, 2, 1):
                    acc = acc + _lane_shuffle(acc, lane ^ d)
                return jnp.where(lane == j, acc, acc_out)

            accs = lax.fori_loop(
                0, LANES, row_body, jnp.zeros((LANES,), jnp.float32)
            )
            outv[pl.ds(g * LANES, LANES)] = 1.0 / (1.0 + jnp.exp(-accs))
            return carry

        lax.fori_loop(0, CHUNK // LANES, group_body, 0)
        pltpu.sync_copy(outv, out.at[pl.ds(base, CHUNK)])

    issue(0, slots[0])
    for c in range(NCHUNK):
        if c + 1 < NCHUNK:
            issue(c + 1, slots[(c + 1) % NSLOT])
        drain(slots[c % NSLOT])
        compute(c, slots[c % NSLOT])


def kernel(X, s_embeds, r_embeds, w_embeds):
    xT = X.astype(jnp.int32).T
    return _sc_forward(xT, s_embeds, r_embeds, w_embeds)
